# baseline (speedup 1.0000x reference)
import functools

import jax
import jax.numpy as jnp
from jax import lax
from jax.experimental import pallas as pl
from jax.experimental.pallas import tpu as pltpu
from jax.experimental.pallas import tpu_sc as plsc

H = 64
W = 64
C = 3
KER = 5
PAD = 10
NKEYS = 8192
D = C * KER * KER
HO = H + 2 * PAD - KER + 1
L = HO * HO
DP = 128

BM = 1600
BK = 512
MT = L // BM
KT = NKEYS // BK

SC_CORES = 2
SC_SUBCORES = 16
NW = SC_CORES * SC_SUBCORES
BPW = L // NW


def _argmin_body(q_ref, keys_ref, q2_ref, k2_ref, idx_ref, minv, mini):
    k = pl.program_id(1)
    g = lax.dot_general(q_ref[...], keys_ref[...], (((1,), (1,)), ((), ())),
                        preferred_element_type=jnp.float32)
    dist = (q2_ref[...] - g) + k2_ref[0:1, :]
    lmin = jnp.min(dist, axis=1, keepdims=True)
    cols = lax.broadcasted_iota(jnp.int32, dist.shape, 1)
    lidx = (jnp.min(jnp.where(dist == lmin, cols, NKEYS), axis=1, keepdims=True)
            + k * BK)

    @pl.when(k == 0)
    def _():
        minv[...] = lmin
        mini[...] = lidx

    @pl.when(k > 0)
    def _():
        better = lmin < minv[...]
        mini[...] = jnp.where(better, lidx, mini[...])
        minv[...] = jnp.where(better, lmin, minv[...])

    @pl.when(k == KT - 1)
    def _():
        idx_ref[...] = mini[...]


def _nearest_idx(q2x, keys, q2, k2):
    return pl.pallas_call(
        _argmin_body,
        grid=(MT, KT),
        in_specs=[
            pl.BlockSpec((BM, D), lambda m, k: (m, 0)),
            pl.BlockSpec((BK, D), lambda m, k: (k, 0)),
            pl.BlockSpec((BM, 1), lambda m, k: (m, 0)),
            pl.BlockSpec((8, BK), lambda m, k: (0, k)),
        ],
        out_specs=pl.BlockSpec((BM, 1), lambda m, k: (m, 0)),
        out_shape=jax.ShapeDtypeStruct((L, 1), jnp.int32),
        scratch_shapes=[
            pltpu.VMEM((BM, 1), jnp.float32),
            pltpu.VMEM((BM, 1), jnp.int32),
        ],
    )(q2x, keys, q2, k2)


def _sc_gather(keys_pad, idx):
    mesh = plsc.VectorSubcoreMesh(core_axis_name="c", subcore_axis_name="s")

    @functools.partial(
        pl.kernel,
        mesh=mesh,
        out_type=jax.ShapeDtypeStruct((L, DP), jnp.float32),
        scratch_types=[
            pltpu.VMEM((BPW,), jnp.int32),
            pltpu.VMEM((BPW, DP), jnp.float32),
            pltpu.SemaphoreType.DMA,
        ],
    )
    def gather(table_hbm, idx_hbm, out_hbm, idx_v, rows_v, sem):
        wid = lax.axis_index("s") * SC_CORES + lax.axis_index("c")
        base = wid * BPW
        pltpu.sync_copy(idx_hbm.at[pl.ds(base, BPW)], idx_v)
        pltpu.async_copy(table_hbm.at[idx_v], rows_v, sem).wait()
        pltpu.sync_copy(rows_v, out_hbm.at[pl.ds(base, BPW)])

    return gather(keys_pad, idx)


def _fold_body(x_ref, out_ref):
    chans = []
    for c in range(C):
        acc = jnp.zeros((H, W), jnp.float32)
        for di in range(KER):
            for dj in range(KER):
                p = x_ref[c * KER * KER + di * KER + dj]
                acc = acc + p[PAD - di:PAD - di + H, PAD - dj:PAD - dj + W]
        chans.append(acc)
    stack = jnp.stack(chans, axis=0)
    out_ref[...] = stack / jnp.max(stack)


def _fold(x):
    return pl.pallas_call(
        _fold_body,
        out_shape=jax.ShapeDtypeStruct((C, H, W), jnp.float32),
    )(x)


def kernel(image, keys):
    img = jnp.transpose(image, (2, 0, 1))
    xp = jnp.pad(img, ((0, 0), (PAD, PAD), (PAD, PAD)))
    patches = jnp.stack(
        [xp[:, di:di + HO, dj:dj + HO] for di in range(KER) for dj in range(KER)],
        axis=1)
    q = patches.reshape(D, L).T
    q2 = jnp.sum(q * q, axis=1, keepdims=True)
    k2 = jnp.broadcast_to(jnp.sum(keys * keys, axis=1)[None, :], (8, NKEYS))

    idx = _nearest_idx(2.0 * q, keys, q2, k2).reshape(L)
    keys_pad = jnp.pad(keys, ((0, 0), (0, DP - D)))
    recon = _sc_gather(keys_pad, idx)
    x = recon[:, :D].T.reshape(D, HO, HO)
    out = _fold(x)
    return jnp.transpose(out, (1, 2, 0))


# speedup vs baseline: 1.1001x; 1.1001x over previous
import functools

import jax
import jax.numpy as jnp
from jax import lax
from jax.experimental import pallas as pl
from jax.experimental.pallas import tpu as pltpu
from jax.experimental.pallas import tpu_sc as plsc

H = 64
W = 64
C = 3
KER = 5
PAD = 10
NKEYS = 8192
D = C * KER * KER
HO = H + 2 * PAD - KER + 1
L = HO * HO
DP = 128

BM = 1280
BK = 512
MT = L // BM
KT = NKEYS // BK

SC_CORES = 2
SC_SUBCORES = 16
NW = SC_CORES * SC_SUBCORES
BPW = BM // NW


def _argmin_body(qt_ref, keys_ref, q2_ref, k2_ref, idx_ref, minv, mini):
    k = pl.program_id(0)
    g = lax.dot_general(qt_ref[...], keys_ref[...], (((0,), (1,)), ((), ())),
                        preferred_element_type=jnp.float32)
    dist = (q2_ref[...] - g) + k2_ref[0:1, :]
    lmin = jnp.min(dist, axis=1, keepdims=True)
    cols = lax.broadcasted_iota(jnp.int32, dist.shape, 1)
    lidx = (jnp.min(jnp.where(dist == lmin, cols, NKEYS), axis=1, keepdims=True)
            + k * BK)

    @pl.when(k == 0)
    def _():
        minv[...] = lmin
        mini[...] = lidx

    @pl.when(k > 0)
    def _():
        better = lmin < minv[...]
        mini[...] = jnp.where(better, lidx, mini[...])
        minv[...] = jnp.where(better, lmin, minv[...])

    @pl.when(k == KT - 1)
    def _():
        idx_ref[...] = mini[...]


def _nearest_idx_part(qt2x, keys, q2, k2, m):
    return pl.pallas_call(
        _argmin_body,
        grid=(KT,),
        in_specs=[
            pl.BlockSpec((D, BM), lambda k, m=m: (0, m)),
            pl.BlockSpec((BK, D), lambda k: (k, 0)),
            pl.BlockSpec((BM, 1), lambda k, m=m: (m, 0)),
            pl.BlockSpec((8, BK), lambda k: (0, k)),
        ],
        out_specs=pl.BlockSpec((BM, 1), lambda k: (0, 0)),
        out_shape=jax.ShapeDtypeStruct((BM, 1), jnp.int32),
        scratch_shapes=[
            pltpu.VMEM((BM, 1), jnp.float32),
            pltpu.VMEM((BM, 1), jnp.int32),
        ],
    )(qt2x, keys, q2, k2)


def _sc_gather_part(keys_pad, idx):
    mesh = plsc.VectorSubcoreMesh(core_axis_name="c", subcore_axis_name="s")

    @functools.partial(
        pl.kernel,
        mesh=mesh,
        out_type=jax.ShapeDtypeStruct((BM, DP), jnp.float32),
        scratch_types=[
            pltpu.VMEM((BPW,), jnp.int32),
            pltpu.VMEM((BPW, DP), jnp.float32),
            pltpu.SemaphoreType.DMA,
        ],
    )
    def gather(table_hbm, idx_hbm, out_hbm, idx_v, rows_v, sem):
        wid = lax.axis_index("s") * SC_CORES + lax.axis_index("c")
        base = wid * BPW
        pltpu.sync_copy(idx_hbm.at[pl.ds(base, BPW)], idx_v)
        pltpu.async_copy(table_hbm.at[idx_v], rows_v, sem).wait()
        pltpu.sync_copy(rows_v, out_hbm.at[pl.ds(base, BPW)])

    return gather(keys_pad, idx)


def _fold_body(x_ref, out_ref):
    chans = []
    for c in range(C):
        acc = jnp.zeros((H, W), jnp.float32)
        for di in range(KER):
            for dj in range(KER):
                p = x_ref[c * KER * KER + di * KER + dj]
                acc = acc + p[PAD - di:PAD - di + H, PAD - dj:PAD - dj + W]
        chans.append(acc)
    stack = jnp.stack(chans, axis=0)
    out_ref[...] = stack / jnp.max(stack)


def _fold(x):
    return pl.pallas_call(
        _fold_body,
        out_shape=jax.ShapeDtypeStruct((C, H, W), jnp.float32),
    )(x)


def kernel(image, keys):
    img = jnp.transpose(image, (2, 0, 1))
    xp = jnp.pad(img, ((0, 0), (PAD, PAD), (PAD, PAD)))
    patches = jnp.stack(
        [xp[:, di:di + HO, dj:dj + HO] for di in range(KER) for dj in range(KER)],
        axis=1)
    qt = patches.reshape(D, L)
    q2 = jnp.sum(qt * qt, axis=0)[:, None]
    k2 = jnp.broadcast_to(jnp.sum(keys * keys, axis=1)[None, :], (8, NKEYS))
    keys_pad = jnp.pad(keys, ((0, 0), (0, DP - D)))
    qt2x = 2.0 * qt

    recons = []
    for m in range(MT):
        idx_m = _nearest_idx_part(qt2x, keys, q2, k2, m).reshape(BM)
        recons.append(_sc_gather_part(keys_pad, idx_m))
    recon = jnp.concatenate(recons, axis=0)
    x = recon[:, :D].T.reshape(D, HO, HO)
    out = _fold(x)
    return jnp.transpose(out, (1, 2, 0))


# speedup vs baseline: 1.1350x; 1.0317x over previous
import functools

import jax
import jax.numpy as jnp
from jax import lax
from jax.experimental import pallas as pl
from jax.experimental.pallas import tpu as pltpu
from jax.experimental.pallas import tpu_sc as plsc

H = 64
W = 64
C = 3
KER = 5
PAD = 10
NKEYS = 8192
D = C * KER * KER
HO = H + 2 * PAD - KER + 1
L = HO * HO
DP = 128

BM = 1280
BK = 512
MT = L // BM
KT = NKEYS // BK

SC_CORES = 2
SC_SUBCORES = 16
NW = SC_CORES * SC_SUBCORES
BPW = BM // NW


def _argmin_body(qt_ref, keys_ref, q2_ref, k2_ref, idx_ref, minv, mini):
    k = pl.program_id(0)
    g = lax.dot_general(qt_ref[...], keys_ref[...], (((0,), (1,)), ((), ())),
                        preferred_element_type=jnp.float32)
    dist = (q2_ref[...] - g) + k2_ref[0:1, :]
    lmin = jnp.min(dist, axis=1, keepdims=True)
    cols = lax.broadcasted_iota(jnp.int32, dist.shape, 1)
    lidx = (jnp.min(jnp.where(dist == lmin, cols, NKEYS), axis=1, keepdims=True)
            + k * BK)

    @pl.when(k == 0)
    def _():
        minv[...] = lmin
        mini[...] = lidx

    @pl.when(k > 0)
    def _():
        better = lmin < minv[...]
        mini[...] = jnp.where(better, lidx, mini[...])
        minv[...] = jnp.where(better, lmin, minv[...])

    @pl.when(k == KT - 1)
    def _():
        idx_ref[...] = mini[...].reshape(BM)


def _nearest_idx_part(qt2x, keys, q2, k2, m):
    return pl.pallas_call(
        _argmin_body,
        grid=(KT,),
        in_specs=[
            pl.BlockSpec((D, BM), lambda k, m=m: (0, m)),
            pl.BlockSpec((BK, D), lambda k: (k, 0)),
            pl.BlockSpec((BM, 1), lambda k, m=m: (m, 0)),
            pl.BlockSpec((8, BK), lambda k: (0, k)),
        ],
        out_specs=pl.BlockSpec((BM,), lambda k: (0,)),
        out_shape=jax.ShapeDtypeStruct((BM,), jnp.int32),
        scratch_shapes=[
            pltpu.VMEM((BM, 1), jnp.float32),
            pltpu.VMEM((BM, 1), jnp.int32),
        ],
    )(qt2x, keys, q2, k2)


_SC_MESH = plsc.VectorSubcoreMesh(core_axis_name="c", subcore_axis_name="s")


@functools.partial(
    pl.kernel,
    mesh=_SC_MESH,
    out_type=jax.ShapeDtypeStruct((BM, DP), jnp.float32),
    scratch_types=[
        pltpu.VMEM((BPW,), jnp.int32),
        pltpu.VMEM((BPW, DP), jnp.float32),
        pltpu.SemaphoreType.DMA,
    ],
)
def _sc_gather_part(table_hbm, idx_hbm, out_hbm, idx_v, rows_v, sem):
    wid = lax.axis_index("s") * SC_CORES + lax.axis_index("c")
    base = wid * BPW
    pltpu.sync_copy(idx_hbm.at[pl.ds(base, BPW)], idx_v)
    pltpu.async_copy(table_hbm.at[idx_v], rows_v, sem).wait()
    pltpu.sync_copy(rows_v, out_hbm.at[pl.ds(base, BPW)])


def _fold_body(x_ref, out_ref):
    chans = []
    for c in range(C):
        acc = jnp.zeros((H, W), jnp.float32)
        for di in range(KER):
            for dj in range(KER):
                p = x_ref[c * KER * KER + di * KER + dj]
                acc = acc + p[PAD - di:PAD - di + H, PAD - dj:PAD - dj + W]
        chans.append(acc)
    stack = jnp.stack(chans, axis=0)
    out_ref[...] = stack / jnp.max(stack)


def _fold(x):
    return pl.pallas_call(
        _fold_body,
        out_shape=jax.ShapeDtypeStruct((C, H, W), jnp.float32),
    )(x)


def kernel(image, keys):
    img = jnp.transpose(image, (2, 0, 1))
    xp = jnp.pad(img, ((0, 0), (PAD, PAD), (PAD, PAD)))
    patches = jnp.stack(
        [xp[:, di:di + HO, dj:dj + HO] for di in range(KER) for dj in range(KER)],
        axis=1)
    qt = patches.reshape(D, L)
    q2 = jnp.sum(qt * qt, axis=0)[:, None]
    k2 = jnp.broadcast_to(jnp.sum(keys * keys, axis=1)[None, :], (8, NKEYS))
    keys_pad = jnp.pad(keys, ((0, 0), (0, DP - D)))
    qt2x = 2.0 * qt

    recons = []
    for m in range(MT):
        idx_m = _nearest_idx_part(qt2x, keys, q2, k2, m)
        recons.append(_sc_gather_part(keys_pad, idx_m))
    recon = jnp.concatenate(recons, axis=0)
    x = recon[:, :D].T.reshape(D, HO, HO)
    out = _fold(x)
    return jnp.transpose(out, (1, 2, 0))


# speedup vs baseline: 1.4689x; 1.2942x over previous
import functools

import jax
import jax.numpy as jnp
from jax import lax
from jax.experimental import pallas as pl
from jax.experimental.pallas import tpu as pltpu
from jax.experimental.pallas import tpu_sc as plsc

H = 64
W = 64
C = 3
KER = 5
PAD = 10
NKEYS = 8192
D = C * KER * KER
HO = H + 2 * PAD - KER + 1
L = HO * HO
DP = 128

BM = 1280
BK = 8192
MT = L // BM
KT = NKEYS // BK

SC_CORES = 2
SC_SUBCORES = 16
NW = SC_CORES * SC_SUBCORES
BPW = BM // NW


def _argmin_body(qt_ref, keys_ref, q2_ref, k2_ref, idx_ref, minv, mini):
    k = pl.program_id(0)
    g = lax.dot_general(qt_ref[...], keys_ref[...], (((0,), (1,)), ((), ())),
                        preferred_element_type=jnp.float32)
    dist = (q2_ref[...] - g) + k2_ref[0:1, :]
    lmin = jnp.min(dist, axis=1, keepdims=True)
    cols = lax.broadcasted_iota(jnp.int32, dist.shape, 1)
    lidx = (jnp.min(jnp.where(dist == lmin, cols, NKEYS), axis=1, keepdims=True)
            + k * BK)

    @pl.when(k == 0)
    def _():
        minv[...] = lmin
        mini[...] = lidx

    @pl.when(k > 0)
    def _():
        better = lmin < minv[...]
        mini[...] = jnp.where(better, lidx, mini[...])
        minv[...] = jnp.where(better, lmin, minv[...])

    @pl.when(k == KT - 1)
    def _():
        idx_ref[...] = mini[...].reshape(BM)


def _nearest_idx_part(qt2x, keys, q2, k2, m):
    return pl.pallas_call(
        _argmin_body,
        grid=(KT,),
        in_specs=[
            pl.BlockSpec((D, BM), lambda k, m=m: (0, m)),
            pl.BlockSpec((BK, D), lambda k: (k, 0)),
            pl.BlockSpec((BM, 1), lambda k, m=m: (m, 0)),
            pl.BlockSpec((8, BK), lambda k: (0, k)),
        ],
        out_specs=pl.BlockSpec((BM,), lambda k: (0,)),
        out_shape=jax.ShapeDtypeStruct((BM,), jnp.int32),
        scratch_shapes=[
            pltpu.VMEM((BM, 1), jnp.float32),
            pltpu.VMEM((BM, 1), jnp.int32),
        ],
    )(qt2x, keys, q2, k2)


_SC_MESH = plsc.VectorSubcoreMesh(core_axis_name="c", subcore_axis_name="s")


@functools.partial(
    pl.kernel,
    mesh=_SC_MESH,
    out_type=jax.ShapeDtypeStruct((BM, DP), jnp.float32),
    scratch_types=[
        pltpu.VMEM((BPW,), jnp.int32),
        pltpu.VMEM((BPW, DP), jnp.float32),
        pltpu.SemaphoreType.DMA,
    ],
)
def _sc_gather_part(table_hbm, idx_hbm, out_hbm, idx_v, rows_v, sem):
    wid = lax.axis_index("s") * SC_CORES + lax.axis_index("c")
    base = wid * BPW
    pltpu.sync_copy(idx_hbm.at[pl.ds(base, BPW)], idx_v)
    pltpu.async_copy(table_hbm.at[idx_v], rows_v, sem).wait()
    pltpu.sync_copy(rows_v, out_hbm.at[pl.ds(base, BPW)])


def _fold_body(x_ref, out_ref):
    chans = []
    for c in range(C):
        acc = jnp.zeros((H, W), jnp.float32)
        for di in range(KER):
            for dj in range(KER):
                p = x_ref[c * KER * KER + di * KER + dj]
                acc = acc + p[PAD - di:PAD - di + H, PAD - dj:PAD - dj + W]
        chans.append(acc)
    stack = jnp.stack(chans, axis=0)
    out_ref[...] = stack / jnp.max(stack)


def _fold(x):
    return pl.pallas_call(
        _fold_body,
        out_shape=jax.ShapeDtypeStruct((C, H, W), jnp.float32),
    )(x)


def kernel(image, keys):
    img = jnp.transpose(image, (2, 0, 1))
    xp = jnp.pad(img, ((0, 0), (PAD, PAD), (PAD, PAD)))
    patches = jnp.stack(
        [xp[:, di:di + HO, dj:dj + HO] for di in range(KER) for dj in range(KER)],
        axis=1)
    qt = patches.reshape(D, L)
    q2 = jnp.sum(qt * qt, axis=0)[:, None]
    k2 = jnp.broadcast_to(jnp.sum(keys * keys, axis=1)[None, :], (8, NKEYS))
    keys_pad = jnp.pad(keys, ((0, 0), (0, DP - D)))
    qt2x = 2.0 * qt

    recons = []
    for m in range(MT):
        idx_m = _nearest_idx_part(qt2x, keys, q2, k2, m)
        recons.append(_sc_gather_part(keys_pad, idx_m))
    recon = jnp.concatenate(recons, axis=0)
    x = recon[:, :D].T.reshape(D, HO, HO)
    out = _fold(x)
    return jnp.transpose(out, (1, 2, 0))


# speedup vs baseline: 1.4692x; 1.0001x over previous
import functools

import jax
import jax.numpy as jnp
from jax import lax
from jax.experimental import pallas as pl
from jax.experimental.pallas import tpu as pltpu
from jax.experimental.pallas import tpu_sc as plsc

H = 64
W = 64
C = 3
KER = 5
PAD = 10
NKEYS = 8192
D = C * KER * KER
HO = H + 2 * PAD - KER + 1
L = HO * HO
DP = 128

BM = 1280
BK = 8192
MT = L // BM
KT = NKEYS // BK

SC_CORES = 2
SC_SUBCORES = 16
NW = SC_CORES * SC_SUBCORES
BPW = BM // NW


def _argmin_body(qt_ref, keys_ref, q2_ref, k2_ref, idx_ref, minv, mini):
    k = pl.program_id(0)
    g = lax.dot_general(qt_ref[...], keys_ref[...], (((0,), (1,)), ((), ())),
                        preferred_element_type=jnp.float32)
    dist = (q2_ref[...] - g) + k2_ref[0:1, :]
    lmin = jnp.min(dist, axis=1, keepdims=True)
    cols = lax.broadcasted_iota(jnp.int32, dist.shape, 1)
    lidx = (jnp.min(jnp.where(dist == lmin, cols, NKEYS), axis=1, keepdims=True)
            + k * BK)

    @pl.when(k == 0)
    def _():
        minv[...] = lmin
        mini[...] = lidx

    @pl.when(k > 0)
    def _():
        better = lmin < minv[...]
        mini[...] = jnp.where(better, lidx, mini[...])
        minv[...] = jnp.where(better, lmin, minv[...])

    @pl.when(k == KT - 1)
    def _():
        idx_ref[...] = mini[...].reshape(BM)


def _nearest_idx_part(qt2x, keys, q2, k2, m):
    return pl.pallas_call(
        _argmin_body,
        grid=(KT,),
        in_specs=[
            pl.BlockSpec((D, BM), lambda k, m=m: (0, m)),
            pl.BlockSpec((BK, D), lambda k: (k, 0)),
            pl.BlockSpec((BM, 1), lambda k, m=m: (m, 0)),
            pl.BlockSpec((8, BK), lambda k: (0, k)),
        ],
        out_specs=pl.BlockSpec((BM,), lambda k: (0,)),
        out_shape=jax.ShapeDtypeStruct((BM,), jnp.int32),
        scratch_shapes=[
            pltpu.VMEM((BM, 1), jnp.float32),
            pltpu.VMEM((BM, 1), jnp.int32),
        ],
    )(qt2x, keys, q2, k2)


_SC_MESH = plsc.VectorSubcoreMesh(core_axis_name="c", subcore_axis_name="s")


@functools.partial(
    pl.kernel,
    mesh=_SC_MESH,
    out_type=jax.ShapeDtypeStruct((BM, DP), jnp.float32),
    scratch_types=[
        pltpu.VMEM((BPW,), jnp.int32),
        pltpu.VMEM((BPW, DP), jnp.float32),
        pltpu.SemaphoreType.DMA,
    ],
)
def _sc_gather_part(table_hbm, idx_hbm, out_hbm, idx_v, rows_v, sem):
    wid = lax.axis_index("s") * SC_CORES + lax.axis_index("c")
    base = wid * BPW
    pltpu.sync_copy(idx_hbm.at[pl.ds(base, BPW)], idx_v)
    pltpu.async_copy(table_hbm.at[idx_v], rows_v, sem).wait()
    pltpu.sync_copy(rows_v, out_hbm.at[pl.ds(base, BPW)])


def _fold_body(x_ref, out_ref):
    chans = []
    for c in range(C):
        acc = jnp.zeros((H, W), jnp.float32)
        for di in range(KER):
            for dj in range(KER):
                p = x_ref[c * KER * KER + di * KER + dj]
                acc = acc + p[PAD - di:PAD - di + H, PAD - dj:PAD - dj + W]
        chans.append(acc)
    stack = jnp.stack(chans, axis=0)
    out_ref[...] = stack / jnp.max(stack)


def _fold(x):
    return pl.pallas_call(
        _fold_body,
        out_shape=jax.ShapeDtypeStruct((C, H, W), jnp.float32),
    )(x)


def kernel(image, keys):
    img = jnp.transpose(image, (2, 0, 1))
    xp = jnp.pad(img, ((0, 0), (PAD, PAD), (PAD, PAD)))
    patches = jnp.stack(
        [xp[:, di:di + HO, dj:dj + HO] for di in range(KER) for dj in range(KER)],
        axis=1)
    qt = patches.reshape(D, L)
    q2 = jnp.sum(qt * qt, axis=0)[:, None]
    k2 = jnp.broadcast_to(jnp.sum(keys * keys, axis=1)[None, :], (8, NKEYS))
    keys_pad = jnp.pad(keys, ((0, 0), (0, DP - D)))
    qt2x = 2.0 * qt

    recons = [None] * MT
    for m in (1, 0, 2, 4, 3):
        idx_m = _nearest_idx_part(qt2x, keys, q2, k2, m)
        recons[m] = _sc_gather_part(keys_pad, idx_m)
    recon = jnp.concatenate(recons, axis=0)
    x = recon[:, :D].T.reshape(D, HO, HO)
    out = _fold(x)
    return jnp.transpose(out, (1, 2, 0))
